# baseline (speedup 1.0000x reference)
import functools

import jax
import jax.numpy as jnp
from jax import lax
from jax.experimental import pallas as pl
from jax.experimental.pallas import tpu as pltpu
from jax.experimental.pallas import tpu_sc as plsc

NC, NS, L = 2, 16, 16
NW = NC * NS
CHUNK = 128


@functools.partial(jax.jit, static_argnums=(7, 8, 9))
def _glove_sc(cw, tw, co, wt, big, vb, ub, B, D, Vhalf):
    n_per_w = B // NW
    n_chunks = n_per_w // CHUNK
    n_groups = CHUNK // L
    D2 = 2 * D

    mesh = plsc.VectorSubcoreMesh(core_axis_name="c", subcore_axis_name="s")

    @functools.partial(
        pl.kernel,
        out_type=jax.ShapeDtypeStruct((NW, L), jnp.float32),
        mesh=mesh,
        compiler_params=pltpu.CompilerParams(needs_layout_passes=False),
        scratch_types=[
            pltpu.VMEM((n_per_w,), jnp.int32),
            pltpu.VMEM((n_per_w,), jnp.int32),
            pltpu.VMEM((n_per_w,), jnp.int32),
            pltpu.VMEM((n_per_w,), jnp.int32),
            pltpu.VMEM((2, CHUNK, D2), jnp.float32),
            pltpu.VMEM((2, CHUNK, D2), jnp.float32),
            pltpu.VMEM((n_per_w,), jnp.float32),
            pltpu.VMEM((n_per_w,), jnp.float32),
            pltpu.VMEM((n_per_w,), jnp.float32),
            pltpu.VMEM((n_per_w,), jnp.float32),
            pltpu.VMEM((L,), jnp.float32),
            pltpu.SemaphoreType.DMA,
            pltpu.SemaphoreType.DMA,
            pltpu.SemaphoreType.DMA,
        ],
    )
    def glove_kernel(cw_hbm, tw_hbm, co_hbm, wt_hbm, big_hbm,
                     vb_hbm, ub_hbm, out_hbm,
                     idxc, idxt, sidxc, sidxt, rowsc, rowst,
                     cb, tb, cov, wv, obuf, bsem, sem0, sem1):
        wid = lax.axis_index("c") * NS + lax.axis_index("s")
        base = wid * n_per_w
        sems = (sem0, sem1)

        pltpu.sync_copy(cw_hbm.at[pl.ds(base, n_per_w)], idxc)
        pltpu.sync_copy(tw_hbm.at[pl.ds(base, n_per_w)], idxt)
        pltpu.sync_copy(co_hbm.at[pl.ds(base, n_per_w)], cov)
        pltpu.sync_copy(wt_hbm.at[pl.ds(base, n_per_w)], wv)

        def shift(g, _):
            gsl = pl.ds(g * L, L)
            sidxc[gsl] = lax.shift_right_logical(idxc[gsl], 1)
            sidxt[gsl] = lax.shift_right_logical(idxt[gsl], 1) + Vhalf
            return 0

        lax.fori_loop(0, n_per_w // L, shift, 0)

        bias_copies = []
        for j in range(n_chunks):
            dst = pl.ds(j * CHUNK, CHUNK)
            bias_copies.append(pltpu.async_copy(vb_hbm.at[idxc.at[dst]],
                                                cb.at[dst], bsem))
            bias_copies.append(pltpu.async_copy(ub_hbm.at[idxt.at[dst]],
                                                tb.at[dst], bsem))

        def fire(j, buf):
            src = pl.ds(j * CHUNK, CHUNK)
            pltpu.async_copy(big_hbm.at[sidxc.at[src]], rowsc.at[buf],
                             sems[buf])
            pltpu.async_copy(big_hbm.at[sidxt.at[src]], rowst.at[buf],
                             sems[buf])

        def drain(buf):
            pltpu.make_async_copy(big_hbm.at[pl.ds(0, CHUNK)], rowsc.at[buf],
                                  sems[buf]).wait()
            pltpu.make_async_copy(big_hbm.at[pl.ds(0, CHUNK)], rowst.at[buf],
                                  sems[buf]).wait()

        fire(0, 0)
        for c in bias_copies:
            c.wait()

        lane = lax.iota(jnp.int32, L)
        acc = jnp.zeros((L,), jnp.float32)
        for j in range(n_chunks):
            buf = j % 2
            if j + 1 < n_chunks:
                fire(j + 1, 1 - buf)
            drain(buf)

            def group(g, a):
                goff = g * L
                vparc = (idxc[pl.ds(j * CHUNK + goff, L)] & 1) * D
                vpart = (idxt[pl.ds(j * CHUNK + goff, L)] & 1) * D
                ips = jnp.zeros((L,), jnp.float32)
                for i in range(L):
                    offc = vparc[i]
                    offt = vpart[i]
                    p = goff + i
                    s = (rowsc[buf, p, pl.ds(offc, L)]
                         * rowst[buf, p, pl.ds(offt, L)])
                    for k in range(1, D // L):
                        s = s + (rowsc[buf, p, pl.ds(offc + k * L, L)]
                                 * rowst[buf, p, pl.ds(offt + k * L, L)])
                    ips = jnp.where(lane == i, jnp.sum(s), ips)
                gsl = pl.ds(j * CHUNK + goff, L)
                err = ips + cb[gsl] + tb[gsl] - cov[gsl]
                return a + wv[gsl] * err * err

            acc = lax.fori_loop(0, n_groups, group, acc)

        obuf[...] = acc
        pltpu.sync_copy(obuf, out_hbm.at[wid])

    return glove_kernel(cw, tw, co, wt, big, vb, ub)


def kernel(center_words, target_words, coocs, weighting,
           emb_v, emb_u, v_bias, u_bias):
    B = center_words.shape[0]
    V, D = emb_v.shape
    big = jnp.concatenate([emb_v, emb_u], axis=0).reshape(V, 2 * D)
    partials = _glove_sc(center_words.reshape(-1).astype(jnp.int32),
                         target_words.reshape(-1).astype(jnp.int32),
                         coocs.reshape(-1), weighting.reshape(-1),
                         big, v_bias.reshape(-1), u_bias.reshape(-1),
                         B, D, V // 2)
    return jnp.sum(partials)


# speedup vs baseline: 1.3168x; 1.3168x over previous
import functools

import jax
import jax.numpy as jnp
from jax import lax
from jax.experimental import pallas as pl
from jax.experimental.pallas import tpu as pltpu
from jax.experimental.pallas import tpu_sc as plsc

NC, NS, L = 2, 16, 16
NW = NC * NS
CHUNK = 128


@functools.partial(jax.jit, static_argnums=(8, 9))
def _glove_sc(cw, tw, co, wt, emb_v, emb_u, vb, ub, B, D):
    n_per_w = B // NW
    n_chunks = n_per_w // CHUNK
    n_groups = n_per_w // L

    mesh = plsc.VectorSubcoreMesh(core_axis_name="c", subcore_axis_name="s")

    @functools.partial(
        pl.kernel,
        out_type=jax.ShapeDtypeStruct((NW, L), jnp.float32),
        mesh=mesh,
        compiler_params=pltpu.CompilerParams(needs_layout_passes=False,
                                             use_tc_tiling_on_sc=False),
        scratch_types=[
            pltpu.VMEM((n_chunks, CHUNK), jnp.int32),
            pltpu.VMEM((n_chunks, CHUNK), jnp.int32),
            pltpu.VMEM((n_per_w, D), jnp.float32),
            pltpu.VMEM((n_per_w, D), jnp.float32),
            pltpu.VMEM((n_per_w,), jnp.float32),
            pltpu.VMEM((n_per_w,), jnp.float32),
            pltpu.VMEM((n_per_w,), jnp.float32),
            pltpu.VMEM((n_per_w,), jnp.float32),
            pltpu.VMEM((L,), jnp.float32),
            pltpu.SemaphoreType.DMA,
        ],
    )
    def glove_kernel(cw_hbm, tw_hbm, co_hbm, wt_hbm, ev_hbm, eu_hbm,
                     vb_hbm, ub_hbm, out_hbm,
                     idxc, idxt, rowsc, rowst, cb, tb, cov, wv,
                     obuf, sem):
        wid = lax.axis_index("c") * NS + lax.axis_index("s")
        base = wid * n_per_w
        crow = wid * n_chunks

        pltpu.sync_copy(cw_hbm.at[pl.ds(crow, n_chunks)], idxc)
        pltpu.sync_copy(tw_hbm.at[pl.ds(crow, n_chunks)], idxt)
        pltpu.sync_copy(co_hbm.at[pl.ds(base, n_per_w)], cov)
        pltpu.sync_copy(wt_hbm.at[pl.ds(base, n_per_w)], wv)

        copies = []
        for j in range(n_chunks):
            dst = pl.ds(j * CHUNK, CHUNK)
            copies.append(pltpu.async_copy(ev_hbm.at[idxc.at[j]],
                                           rowsc.at[dst], sem))
            copies.append(pltpu.async_copy(eu_hbm.at[idxt.at[j]],
                                           rowst.at[dst], sem))
            copies.append(pltpu.async_copy(vb_hbm.at[idxc.at[j]],
                                           cb.at[dst], sem))
            copies.append(pltpu.async_copy(ub_hbm.at[idxt.at[j]],
                                           tb.at[dst], sem))
        for c in copies:
            c.wait()

        lane = lax.iota(jnp.int32, L)

        def group(g, acc):
            ips = jnp.zeros((L,), jnp.float32)
            for i in range(L):
                p = g * L + i
                s = rowsc[p, pl.ds(0, L)] * rowst[p, pl.ds(0, L)]
                for k in range(1, D // L):
                    s = s + (rowsc[p, pl.ds(k * L, L)]
                             * rowst[p, pl.ds(k * L, L)])
                ips = jnp.where(lane == i, jnp.sum(s), ips)
            gsl = pl.ds(g * L, L)
            err = ips + cb[gsl] + tb[gsl] - cov[gsl]
            return acc + wv[gsl] * err * err

        acc = lax.fori_loop(0, n_groups, group, jnp.zeros((L,), jnp.float32))
        obuf[...] = acc
        pltpu.sync_copy(obuf, out_hbm.at[wid])

    return glove_kernel(cw, tw, co, wt, emb_v, emb_u, vb, ub)


def kernel(center_words, target_words, coocs, weighting,
           emb_v, emb_u, v_bias, u_bias):
    B = center_words.shape[0]
    D = emb_v.shape[1]
    cw = center_words.reshape(B // CHUNK, CHUNK).astype(jnp.int32)
    tw = target_words.reshape(B // CHUNK, CHUNK).astype(jnp.int32)
    partials = _glove_sc(cw, tw, coocs.reshape(-1), weighting.reshape(-1),
                         emb_v, emb_u, v_bias.reshape(-1), u_bias.reshape(-1),
                         B, D)
    return jnp.sum(partials)
